# baseline (device time: 139811 ns/iter reference)
import jax
import jax.numpy as jnp
from jax import lax
from jax.experimental import pallas as pl
from jax.experimental.pallas import tpu as pltpu

N_DEV = 4
KI = 4


def _fused_body(
    perm_ref,
    x_ref,
    w_ref,
    yex_ref,
    gmax_ref,
    ybuf_ref,
    amax_smem,
    amax_tile,
    amax_all,
    dsend, drecv,
    asend, arecv,
    copy_sem,
):
    j = pl.program_id(0)
    ki = pl.program_id(1)
    me = lax.axis_index("i")
    m_per = x_ref.shape[0]
    bn = w_ref.shape[1]

    def _data_rdma(chunk, q):
        return pltpu.make_async_remote_copy(
            src_ref=ybuf_ref.at[chunk, :, pl.ds(q * bn, bn)],
            dst_ref=yex_ref.at[pl.ds(me * m_per, m_per), pl.ds(q * bn, bn)],
            send_sem=dsend.at[chunk * KI + q],
            recv_sem=drecv.at[chunk * KI + q],
            device_id=(perm_ref[chunk],),
            device_id_type=pl.DeviceIdType.MESH,
        )

    @pl.when(jnp.logical_and(j == 0, ki == 0))
    def _():
        barrier_sem = pltpu.get_barrier_semaphore()
        for d in range(1, N_DEV):
            peer = lax.rem(me + d, N_DEV)
            pl.semaphore_signal(
                barrier_sem, inc=1,
                device_id=(peer,), device_id_type=pl.DeviceIdType.MESH,
            )
        pl.semaphore_wait(barrier_sem, N_DEV - 1)
        amax_smem[0] = 0.0

    y = jnp.dot(
        x_ref[...],
        w_ref[...].astype(jnp.bfloat16),
        preferred_element_type=jnp.float32,
    )
    amax_smem[0] = jnp.maximum(amax_smem[0], jnp.max(jnp.abs(y)))
    ybuf_ref[j, :, pl.ds(ki * bn, bn)] = y.astype(jnp.bfloat16)

    @pl.when(j < N_DEV - 1)
    def _():
        _data_rdma(j, ki).start()

    @pl.when(jnp.logical_and(ki == KI - 1, j == N_DEV - 1))
    def _():
        cp = pltpu.make_async_copy(
            ybuf_ref.at[j],
            yex_ref.at[pl.ds(me * m_per, m_per), :],
            copy_sem,
        )
        cp.start()

        amax_tile[...] = jnp.full_like(amax_tile, amax_smem[0])
        a_rdmas = []
        for d in range(1, N_DEV):
            peer = lax.rem(me + d, N_DEV)
            ar = pltpu.make_async_remote_copy(
                src_ref=amax_tile,
                dst_ref=amax_all.at[d],
                send_sem=asend.at[d],
                recv_sem=arecv.at[d],
                device_id=(peer,),
                device_id_type=pl.DeviceIdType.MESH,
            )
            ar.start()
            a_rdmas.append(ar)

        cp.wait()
        for c in range(N_DEV - 1):
            for q in range(KI):
                _data_rdma(c, q).wait_send()
        for c in range(N_DEV - 1):
            for q in range(KI):
                _data_rdma(c, q).wait_recv()
        for ar in a_rdmas:
            ar.wait()

        g = amax_smem[0]
        for d in range(1, N_DEV):
            g = jnp.maximum(g, amax_all[d, 0, 0])
        gmax_ref[...] = jnp.full_like(gmax_ref, g)


def _fused_gemm_a2a(perm, x, w):
    m_per, k = x.shape
    _, n = w.shape
    n_per = n // N_DEV
    bn = n_per // KI
    grid_spec = pltpu.PrefetchScalarGridSpec(
        num_scalar_prefetch=1,
        grid=(N_DEV, KI),
        in_specs=[
            pl.BlockSpec((m_per, k), lambda j, ki, perm: (0, 0)),
            pl.BlockSpec((k, bn), lambda j, ki, perm: (0, perm[j] * KI + ki)),
        ],
        out_specs=[
            pl.BlockSpec(memory_space=pl.ANY),
            pl.BlockSpec((8, 128), lambda j, ki, perm: (0, 0)),
        ],
        scratch_shapes=[
            pltpu.VMEM((N_DEV, m_per, n_per), jnp.bfloat16),
            pltpu.SMEM((1,), jnp.float32),
            pltpu.VMEM((8, 128), jnp.float32),
            pltpu.VMEM((N_DEV, 8, 128), jnp.float32),
            pltpu.SemaphoreType.DMA((KI * (N_DEV - 1),)),
            pltpu.SemaphoreType.DMA((KI * (N_DEV - 1),)),
            pltpu.SemaphoreType.DMA((N_DEV,)),
            pltpu.SemaphoreType.DMA((N_DEV,)),
            pltpu.SemaphoreType.DMA,
        ],
    )
    return pl.pallas_call(
        _fused_body,
        grid_spec=grid_spec,
        out_shape=[
            jax.ShapeDtypeStruct((N_DEV * m_per, n_per), jnp.bfloat16),
            jax.ShapeDtypeStruct((8, 128), jnp.float32),
        ],
        compiler_params=pltpu.CompilerParams(
            collective_id=0,
            vmem_limit_bytes=60 * 1024 * 1024,
        ),
    )(perm, x, w)


def _quant_body(y_ref, gmax_ref, out_ref):
    g = gmax_ref[0, 0]
    scale = g / 127.0
    y = y_ref[...].astype(jnp.float32)
    q = jnp.clip(jnp.round(y * (127.0 / g)), -127.0, 127.0)
    out_ref[...] = q * scale


def _quant(y_ex, gmax):
    m, n_per = y_ex.shape
    nb = 8
    bm = m // nb
    return pl.pallas_call(
        _quant_body,
        grid=(nb,),
        in_specs=[
            pl.BlockSpec((bm, n_per), lambda j: (j, 0)),
            pl.BlockSpec((8, 128), lambda j: (0, 0)),
        ],
        out_specs=pl.BlockSpec((bm, n_per), lambda j: (j, 0)),
        out_shape=jax.ShapeDtypeStruct((m, n_per), jnp.float32),
    )(y_ex, gmax)


def kernel(x, w_mat):
    me = lax.axis_index("i")
    perm = lax.rem(me + jnp.array([2, 3, 1, 0], dtype=jnp.int32), N_DEV)
    y_ex, gmax = _fused_gemm_a2a(perm, x.astype(jnp.bfloat16), w_mat)
    return _quant(y_ex, gmax)


# device time: 128025 ns/iter; 1.0921x vs baseline; 1.0921x over previous
import jax
import jax.numpy as jnp
from jax import lax
from jax.experimental import pallas as pl
from jax.experimental.pallas import tpu as pltpu

N_DEV = 4
KI = 8


def _fused_body(
    perm_ref,
    x_ref,
    w_ref,
    yex_ref,
    gmax_ref,
    ybuf_ref,
    amax_smem,
    amax_tile,
    amax_all,
    dsend, drecv,
    asend, arecv,
    copy_sem,
):
    j = pl.program_id(0)
    ki = pl.program_id(1)
    me = lax.axis_index("i")
    m_per = x_ref.shape[0]
    bn = w_ref.shape[1]

    def _data_rdma(chunk, q):
        return pltpu.make_async_remote_copy(
            src_ref=ybuf_ref.at[chunk, :, pl.ds(q * bn, bn)],
            dst_ref=yex_ref.at[pl.ds(me * m_per, m_per), pl.ds(q * bn, bn)],
            send_sem=dsend.at[chunk * KI + q],
            recv_sem=drecv.at[chunk * KI + q],
            device_id=(perm_ref[chunk],),
            device_id_type=pl.DeviceIdType.MESH,
        )

    @pl.when(jnp.logical_and(j == 0, ki == 0))
    def _():
        barrier_sem = pltpu.get_barrier_semaphore()
        for d in range(1, N_DEV):
            peer = lax.rem(me + d, N_DEV)
            pl.semaphore_signal(
                barrier_sem, inc=1,
                device_id=(peer,), device_id_type=pl.DeviceIdType.MESH,
            )
        pl.semaphore_wait(barrier_sem, N_DEV - 1)
        amax_smem[0] = 0.0

    y = jnp.dot(
        x_ref[...].astype(jnp.bfloat16),
        w_ref[...].astype(jnp.bfloat16),
        preferred_element_type=jnp.float32,
    )
    amax_smem[0] = jnp.maximum(amax_smem[0], jnp.max(jnp.abs(y)))
    ybuf_ref[j, :, pl.ds(ki * bn, bn)] = y.astype(jnp.bfloat16)

    @pl.when(j < N_DEV - 1)
    def _():
        _data_rdma(j, ki).start()

    @pl.when(jnp.logical_and(ki == KI - 1, j == N_DEV - 1))
    def _():
        cp = pltpu.make_async_copy(
            ybuf_ref.at[j],
            yex_ref.at[pl.ds(me * m_per, m_per), :],
            copy_sem,
        )
        cp.start()

        amax_tile[...] = jnp.full_like(amax_tile, amax_smem[0])
        a_rdmas = []
        for d in range(1, N_DEV):
            peer = lax.rem(me + d, N_DEV)
            ar = pltpu.make_async_remote_copy(
                src_ref=amax_tile,
                dst_ref=amax_all.at[d],
                send_sem=asend.at[d],
                recv_sem=arecv.at[d],
                device_id=(peer,),
                device_id_type=pl.DeviceIdType.MESH,
            )
            ar.start()
            a_rdmas.append(ar)

        cp.wait()
        for c in range(N_DEV - 1):
            for q in range(KI):
                _data_rdma(c, q).wait_send()
        for c in range(N_DEV - 1):
            for q in range(KI):
                _data_rdma(c, q).wait_recv()
        for ar in a_rdmas:
            ar.wait()

        g = amax_smem[0]
        for d in range(1, N_DEV):
            g = jnp.maximum(g, amax_all[d, 0, 0])
        gmax_ref[...] = jnp.full_like(gmax_ref, g)


def _fused_gemm_a2a(perm, x, w):
    m_per, k = x.shape
    _, n = w.shape
    n_per = n // N_DEV
    bn = n_per // KI
    grid_spec = pltpu.PrefetchScalarGridSpec(
        num_scalar_prefetch=1,
        grid=(N_DEV, KI),
        in_specs=[
            pl.BlockSpec((m_per, k), lambda j, ki, perm: (0, 0)),
            pl.BlockSpec((k, bn), lambda j, ki, perm: (0, perm[j] * KI + ki)),
        ],
        out_specs=[
            pl.BlockSpec(memory_space=pl.ANY),
            pl.BlockSpec((8, 128), lambda j, ki, perm: (0, 0)),
        ],
        scratch_shapes=[
            pltpu.VMEM((N_DEV, m_per, n_per), jnp.bfloat16),
            pltpu.SMEM((1,), jnp.float32),
            pltpu.VMEM((8, 128), jnp.float32),
            pltpu.VMEM((N_DEV, 8, 128), jnp.float32),
            pltpu.SemaphoreType.DMA((KI * (N_DEV - 1),)),
            pltpu.SemaphoreType.DMA((KI * (N_DEV - 1),)),
            pltpu.SemaphoreType.DMA((N_DEV,)),
            pltpu.SemaphoreType.DMA((N_DEV,)),
            pltpu.SemaphoreType.DMA,
        ],
    )
    return pl.pallas_call(
        _fused_body,
        grid_spec=grid_spec,
        out_shape=[
            jax.ShapeDtypeStruct((N_DEV * m_per, n_per), jnp.bfloat16),
            jax.ShapeDtypeStruct((8, 128), jnp.float32),
        ],
        compiler_params=pltpu.CompilerParams(
            collective_id=0,
            vmem_limit_bytes=60 * 1024 * 1024,
        ),
    )(perm, x, w)


def _quant_body(y_ref, gmax_ref, out_ref):
    g = gmax_ref[0, 0]
    scale = g / 127.0
    y = y_ref[...].astype(jnp.float32)
    q = jnp.clip(jnp.round(y * (127.0 / g)), -127.0, 127.0)
    out_ref[...] = q * scale


def _quant(y_ex, gmax):
    m, n_per = y_ex.shape
    nb = 8
    bm = m // nb
    return pl.pallas_call(
        _quant_body,
        grid=(nb,),
        in_specs=[
            pl.BlockSpec((bm, n_per), lambda j: (j, 0)),
            pl.BlockSpec((8, 128), lambda j: (0, 0)),
        ],
        out_specs=pl.BlockSpec((bm, n_per), lambda j: (j, 0)),
        out_shape=jax.ShapeDtypeStruct((m, n_per), jnp.float32),
    )(y_ex, gmax)


def kernel(x, w_mat):
    me = lax.axis_index("i")
    perm = lax.rem(me + jnp.array([2, 3, 1, 0], dtype=jnp.int32), N_DEV)
    y_ex, gmax = _fused_gemm_a2a(perm, x, w_mat)
    return _quant(y_ex, gmax)
